# baseline (speedup 1.0000x reference)
import jax
import jax.numpy as jnp
from jax.experimental import pallas as pl
from jax.experimental.pallas import tpu as pltpu

_TB = 4096


def _mlp_kernel(x_ref, w_ref, b_ref, out_ref):
    w0 = w_ref[0:512, :]
    w1 = w_ref[512:1024, 0:256]
    w2 = w_ref[1024:1280, 0:128]
    b0 = b_ref[0:1, :]
    b1 = b_ref[8:9, 0:256]
    b2 = b_ref[16:17, 0:128]
    h = x_ref[...].astype(jnp.bfloat16)
    h = jnp.dot(h, w0, preferred_element_type=jnp.float32)
    h = jnp.maximum(h + b0, 0.0).astype(jnp.bfloat16)
    h = jnp.dot(h, w1, preferred_element_type=jnp.float32)
    h = jnp.maximum(h + b1, 0.0).astype(jnp.bfloat16)
    h = jnp.dot(h, w2, preferred_element_type=jnp.float32)
    out_ref[...] = jnp.maximum(h + b2, 0.0)


def kernel(x, w0, b0, w1, b1, w2, b2):
    x = jax.lax.stop_gradient(x)
    B, Din = x.shape
    d0, d1, d2 = w0.shape[1], w1.shape[1], w2.shape[1]

    bf = jnp.bfloat16
    wbuf = jnp.concatenate([
        w0.astype(bf),
        jnp.pad(w1.astype(bf), ((0, 0), (0, Din - d1))),
        jnp.pad(w2.astype(bf), ((0, 0), (0, Din - d2))),
    ], axis=0)
    bbuf = jnp.concatenate([
        jnp.pad(b0, ((0, 7), (0, Din - d0))),
        jnp.pad(b1, ((0, 7), (0, Din - d1))),
        jnp.pad(b2, ((0, 7), (0, Din - d2))),
    ], axis=0)

    n_tiles = pl.cdiv(B, _TB)
    flops = 2 * B * (Din * d0 + d0 * d1 + d1 * d2)
    bytes_accessed = (B * Din * 4 + B * d2 * 4
                      + wbuf.size * 2 + bbuf.size * 4)
    return pl.pallas_call(
        _mlp_kernel,
        out_shape=jax.ShapeDtypeStruct((B, d2), x.dtype),
        grid=(n_tiles,),
        in_specs=[
            pl.BlockSpec((_TB, Din), lambda i: (i, 0)),
            pl.BlockSpec(wbuf.shape, lambda i: (0, 0)),
            pl.BlockSpec(bbuf.shape, lambda i: (0, 0)),
        ],
        out_specs=pl.BlockSpec((_TB, d2), lambda i: (i, 0)),
        cost_estimate=pl.CostEstimate(
            flops=flops, transcendentals=0, bytes_accessed=bytes_accessed),
        compiler_params=pltpu.CompilerParams(
            dimension_semantics=("parallel",),
            vmem_limit_bytes=64 << 20),
    )(x, wbuf, bbuf)


# speedup vs baseline: 1.0147x; 1.0147x over previous
import jax
import jax.numpy as jnp
from jax.experimental import pallas as pl
from jax.experimental.pallas import tpu as pltpu

_TB = 8192


_SPLIT = 1


def _mlp_kernel(x_ref, w_ref, b_ref, out_ref):
    w0 = w_ref[0:512, :]
    w1 = w_ref[512:1024, 0:256]
    w2 = w_ref[1024:1280, 0:128]
    b0 = b_ref[0:1, :]
    b1 = b_ref[8:9, 0:256]
    b2 = b_ref[16:17, 0:128]
    sr = _TB // _SPLIT
    for s in range(_SPLIT):
        h = x_ref[s * sr:(s + 1) * sr, :].astype(jnp.bfloat16)
        h = jnp.dot(h, w0, preferred_element_type=jnp.float32)
        h = jnp.maximum(h + b0, 0.0).astype(jnp.bfloat16)
        h = jnp.dot(h, w1, preferred_element_type=jnp.float32)
        h = jnp.maximum(h + b1, 0.0).astype(jnp.bfloat16)
        h = jnp.dot(h, w2, preferred_element_type=jnp.float32)
        out_ref[s * sr:(s + 1) * sr, :] = jnp.maximum(h + b2, 0.0)


def kernel(x, w0, b0, w1, b1, w2, b2):
    x = jax.lax.stop_gradient(x)
    B, Din = x.shape
    d0, d1, d2 = w0.shape[1], w1.shape[1], w2.shape[1]

    bf = jnp.bfloat16
    wbuf = jnp.concatenate([
        w0.astype(bf),
        jnp.pad(w1.astype(bf), ((0, 0), (0, Din - d1))),
        jnp.pad(w2.astype(bf), ((0, 0), (0, Din - d2))),
    ], axis=0)
    bbuf = jnp.concatenate([
        jnp.pad(b0, ((0, 7), (0, Din - d0))),
        jnp.pad(b1, ((0, 7), (0, Din - d1))),
        jnp.pad(b2, ((0, 7), (0, Din - d2))),
    ], axis=0)

    n_tiles = pl.cdiv(B, _TB)
    flops = 2 * B * (Din * d0 + d0 * d1 + d1 * d2)
    bytes_accessed = (B * Din * 4 + B * d2 * 4
                      + wbuf.size * 2 + bbuf.size * 4)
    return pl.pallas_call(
        _mlp_kernel,
        out_shape=jax.ShapeDtypeStruct((B, d2), x.dtype),
        grid=(n_tiles,),
        in_specs=[
            pl.BlockSpec((_TB, Din), lambda i: (i, 0)),
            pl.BlockSpec(wbuf.shape, lambda i: (0, 0)),
            pl.BlockSpec(bbuf.shape, lambda i: (0, 0)),
        ],
        out_specs=pl.BlockSpec((_TB, d2), lambda i: (i, 0)),
        cost_estimate=pl.CostEstimate(
            flops=flops, transcendentals=0, bytes_accessed=bytes_accessed),
        compiler_params=pltpu.CompilerParams(
            dimension_semantics=("parallel",),
            vmem_limit_bytes=64 << 20),
    )(x, wbuf, bbuf)
